# Initial kernel scaffold; baseline (speedup 1.0000x reference)
import jax
import jax.numpy as jnp
from jax.experimental import pallas as pl


def kernel(user_idxs, item_idxs, user_idx_tensor, item_idx_tensor, user_scr_tensor, item_scr_tensor, user_emb, item_emb, W1, b1, W2, b2, W3, b3):
    raise NotImplementedError("write your pallas kernel here")


# speedup vs baseline: 1.5429x; 1.5429x over previous
import jax
import jax.numpy as jnp
from jax import lax
from jax.experimental import pallas as pl
from jax.experimental.pallas import tpu as pltpu
from jax.experimental.pallas import tpu_sc as plsc

NC, NS = 2, 16
NW = NC * NS
NEIGH = 50
P = 56
EMB = 16
CH = 128
BLK = 8


def _wid():
    return lax.axis_index("s") * NC + lax.axis_index("c")


def _mesh():
    return plsc.VectorSubcoreMesh(
        core_axis_name="c", subcore_axis_name="s", num_cores=NC, num_subcores=NS)


def _neigh_gather(user_idxs, item_idxs, utab, itab):
    B = user_idxs.shape[0]
    bpw = B // NW

    def body(uidx, iidx, utab_h, itab_h, nu_out, ni_out, idx_v, n_v, sem):
        base = _wid() * bpw
        for idxs, tab, out in ((uidx, utab_h, nu_out), (iidx, itab_h, ni_out)):
            pltpu.sync_copy(idxs.at[pl.ds(base, bpw)], idx_v)
            pltpu.async_copy(tab.at[idx_v], n_v, sem).wait()
            pltpu.sync_copy(n_v, out.at[pl.ds(base, bpw)])

    call = pl.kernel(
        body,
        out_type=(
            jax.ShapeDtypeStruct((B, P), jnp.int32),
            jax.ShapeDtypeStruct((B, P), jnp.int32),
        ),
        mesh=_mesh(),
        scratch_types=[
            pltpu.VMEM((bpw,), jnp.int32),
            pltpu.VMEM((bpw, P), jnp.int32),
            pltpu.SemaphoreType.DMA,
        ],
        compiler_params=pltpu.CompilerParams(use_tc_tiling_on_sc=False),
    )
    return call(user_idxs, item_idxs, utab, itab)


def _row_gather(nu_flat, ni_flat, uscr, iscr, uemb, iemb):
    R = nu_flat.shape[0]
    rpw = R // NW

    def body(nu, ni, uscr_h, iscr_h, uemb_h, iemb_h, su, eu, si, ei,
             nf_v, s_v, e_v, sem_s, sem_e):
        base = _wid() * rpw
        for nf, scr, emb, s_out, e_out in (
                (nu, uscr_h, uemb_h, su, eu), (ni, iscr_h, iemb_h, si, ei)):
            @pl.loop(0, rpw // CH)
            def _(k):
                off = base + k * CH
                pltpu.sync_copy(nf.at[pl.ds(off, CH)], nf_v)
                cs = pltpu.async_copy(scr.at[nf_v], s_v, sem_s)
                ce = pltpu.async_copy(emb.at[nf_v], e_v, sem_e)
                cs.wait()
                ce.wait()
                pltpu.sync_copy(s_v, s_out.at[pl.ds(off, CH)])
                pltpu.sync_copy(e_v, e_out.at[pl.ds(off, CH)])

    call = pl.kernel(
        body,
        out_type=(
            jax.ShapeDtypeStruct((R, P), jnp.float32),
            jax.ShapeDtypeStruct((R, EMB), jnp.float32),
            jax.ShapeDtypeStruct((R, P), jnp.float32),
            jax.ShapeDtypeStruct((R, EMB), jnp.float32),
        ),
        mesh=_mesh(),
        scratch_types=[
            pltpu.VMEM((CH,), jnp.int32),
            pltpu.VMEM((CH, P), jnp.float32),
            pltpu.VMEM((CH, EMB), jnp.float32),
            pltpu.SemaphoreType.DMA,
            pltpu.SemaphoreType.DMA,
        ],
        compiler_params=pltpu.CompilerParams(use_tc_tiling_on_sc=False),
    )
    return call(nu_flat, ni_flat, uscr, iscr, uemb, iemb)


def _mlp_body(su_ref, eu_ref, si_ref, ei_ref, w1_ref, b1_ref, w2_ref, b2_ref,
              w3_ref, b3_ref, out_ref):
    f32 = jnp.float32
    w1 = w1_ref[...]
    gu = jnp.dot(eu_ref[...], w1[0:EMB, :], preferred_element_type=f32)
    gi = jnp.dot(ei_ref[...], w1[EMB:2 * EMB, :], preferred_element_type=f32)
    su = su_ref[...]
    si = si_ref[...]
    pres = []
    for e in range(BLK):
        lo, hi = e * P, (e + 1) * P
        pre = (jnp.dot(su[lo:hi, :], gu[lo:hi, :], preferred_element_type=f32)
               + jnp.dot(si[lo:hi, :], gi[lo:hi, :], preferred_element_type=f32))
        pres.append(pre)
    pre1 = jnp.concatenate(pres, axis=0)
    h1 = jnp.maximum(pre1 + b1_ref[...], 0.0)
    h2 = jnp.maximum(
        jnp.dot(h1, w2_ref[...], preferred_element_type=f32) + b2_ref[...], 0.0)
    z = jnp.dot(h2, w3_ref[...], preferred_element_type=f32) + b3_ref[...]
    o = jax.nn.sigmoid(z)
    r = lax.broadcasted_iota(jnp.int32, (BLK * P, BLK), 0)
    c = lax.broadcasted_iota(jnp.int32, (BLK * P, BLK), 1)
    msk = ((r // P == c) & (r % P < NEIGH)).astype(f32)
    out_ref[0, 0, :] = jnp.sum(o * msk, axis=0) * (1.0 / NEIGH)


def _mlp_call(su, eu, si, ei, W1, b1, W2, b2, W3, b3):
    R = su.shape[0]
    B = R // P
    rows = BLK * P
    out = pl.pallas_call(
        _mlp_body,
        grid=(B // BLK,),
        in_specs=[
            pl.BlockSpec((rows, P), lambda i: (i, 0)),
            pl.BlockSpec((rows, EMB), lambda i: (i, 0)),
            pl.BlockSpec((rows, P), lambda i: (i, 0)),
            pl.BlockSpec((rows, EMB), lambda i: (i, 0)),
            pl.BlockSpec((2 * EMB, EMB), lambda i: (0, 0)),
            pl.BlockSpec((1, EMB), lambda i: (0, 0)),
            pl.BlockSpec((EMB, 8), lambda i: (0, 0)),
            pl.BlockSpec((1, 8), lambda i: (0, 0)),
            pl.BlockSpec((8, 1), lambda i: (0, 0)),
            pl.BlockSpec((1, 1), lambda i: (0, 0)),
        ],
        out_specs=pl.BlockSpec((1, 1, BLK), lambda i: (i, 0, 0)),
        out_shape=jax.ShapeDtypeStruct((B // BLK, 1, BLK), jnp.float32),
    )(su, eu, si, ei, W1, b1.reshape(1, EMB), W2, b2.reshape(1, 8),
      W3, b3.reshape(1, 1))
    return out.reshape(B)


def kernel(user_idxs, item_idxs, user_idx_tensor, item_idx_tensor,
           user_scr_tensor, item_scr_tensor, user_emb, item_emb,
           W1, b1, W2, b2, W3, b3):
    B = user_idxs.shape[0]
    pad = ((0, 0), (0, P - NEIGH))
    utab = jnp.pad(user_idx_tensor, pad)
    itab = jnp.pad(item_idx_tensor, pad)
    uscr = jnp.pad(user_scr_tensor, pad)
    iscr = jnp.pad(item_scr_tensor, pad)
    nu, ni = _neigh_gather(user_idxs, item_idxs, utab, itab)
    su, eu, si, ei = _row_gather(
        nu.reshape(B * P), ni.reshape(B * P), uscr, iscr, user_emb, item_emb)
    return _mlp_call(su, eu, si, ei, W1, b1, W2, b2, W3, b3)


# speedup vs baseline: 1.5622x; 1.0125x over previous
import jax
import jax.numpy as jnp
from jax import lax
from jax.experimental import pallas as pl
from jax.experimental.pallas import tpu as pltpu
from jax.experimental.pallas import tpu_sc as plsc

NC, NS = 2, 16
NW = NC * NS
NEIGH = 50
P = 56
EMB = 16
G = 8
BLK = 8


def _mesh():
    return plsc.VectorSubcoreMesh(
        core_axis_name="c", subcore_axis_name="s", num_cores=NC, num_subcores=NS)


def _gather_all(user_idxs, item_idxs, utab, itab, uscr, iscr, uemb, iemb):
    B = user_idxs.shape[0]
    bpw = B // NW
    ngroups = bpw // G

    def body(uidx, iidx, utab_h, itab_h, uscr_h, iscr_h, uemb_h, iemb_h,
             su, eu, si, ei,
             idx_v, nu_v, ni_v, s_stg, e_stg, sem_n, sem_s, sem_e, sem_o):
        wid = lax.axis_index("s") * NC + lax.axis_index("c")
        ebase = wid * bpw

        pltpu.sync_copy(uidx.at[pl.ds(ebase, bpw)], idx_v.at[0])
        cu = pltpu.async_copy(utab_h.at[idx_v.at[0]], nu_v, sem_n)
        pltpu.sync_copy(iidx.at[pl.ds(ebase, bpw)], idx_v.at[1])
        ci = pltpu.async_copy(itab_h.at[idx_v.at[1]], ni_v, sem_n)
        cu.wait()
        ci.wait()

        for n_v, scr, emb, s_out, e_out in (
                (nu_v, uscr_h, uemb_h, su, eu), (ni_v, iscr_h, iemb_h, si, ei)):
            @pl.loop(0, ngroups)
            def _(g):
                p = lax.rem(g, 2)
                @pl.when(g >= 2)
                def _():
                    pltpu.make_async_copy(
                        s_stg.at[p], s_out.at[pl.ds(0, G * P)], sem_o).wait()
                    pltpu.make_async_copy(
                        e_stg.at[p], e_out.at[pl.ds(0, G * P)], sem_o).wait()
                for m in range(G):
                    e = g * G + m
                    pltpu.async_copy(
                        scr.at[n_v.at[e]], s_stg.at[p, pl.ds(m * P, P)], sem_s)
                    pltpu.async_copy(
                        emb.at[n_v.at[e]], e_stg.at[p, pl.ds(m * P, P)], sem_e)
                for m in range(G):
                    pltpu.make_async_copy(
                        scr.at[pl.ds(0, P)], s_stg.at[p, pl.ds(m * P, P)],
                        sem_s).wait()
                    pltpu.make_async_copy(
                        emb.at[pl.ds(0, P)], e_stg.at[p, pl.ds(m * P, P)],
                        sem_e).wait()
                rbase = (ebase + g * G) * P
                pltpu.async_copy(s_stg.at[p], s_out.at[pl.ds(rbase, G * P)],
                                 sem_o)
                pltpu.async_copy(e_stg.at[p], e_out.at[pl.ds(rbase, G * P)],
                                 sem_o)
            for _p in range(2):
                pltpu.make_async_copy(
                    s_stg.at[_p], s_out.at[pl.ds(0, G * P)], sem_o).wait()
                pltpu.make_async_copy(
                    e_stg.at[_p], e_out.at[pl.ds(0, G * P)], sem_o).wait()

    R = B * P
    call = pl.kernel(
        body,
        out_type=(
            jax.ShapeDtypeStruct((R, P), jnp.float32),
            jax.ShapeDtypeStruct((R, EMB), jnp.float32),
            jax.ShapeDtypeStruct((R, P), jnp.float32),
            jax.ShapeDtypeStruct((R, EMB), jnp.float32),
        ),
        mesh=_mesh(),
        scratch_types=[
            pltpu.VMEM((2, bpw), jnp.int32),
            pltpu.VMEM((bpw, P), jnp.int32),
            pltpu.VMEM((bpw, P), jnp.int32),
            pltpu.VMEM((2, G * P, P), jnp.float32),
            pltpu.VMEM((2, G * P, EMB), jnp.float32),
            pltpu.SemaphoreType.DMA,
            pltpu.SemaphoreType.DMA,
            pltpu.SemaphoreType.DMA,
            pltpu.SemaphoreType.DMA,
        ],
        compiler_params=pltpu.CompilerParams(use_tc_tiling_on_sc=False),
    )
    return call(user_idxs, item_idxs, utab, itab, uscr, iscr, uemb, iemb)


def _mlp_body(su_ref, eu_ref, si_ref, ei_ref, w1_ref, b1_ref, w2_ref, b2_ref,
              w3_ref, b3_ref, out_ref):
    f32 = jnp.float32
    w1 = w1_ref[...]
    gu = jnp.dot(eu_ref[...], w1[0:EMB, :], preferred_element_type=f32)
    gi = jnp.dot(ei_ref[...], w1[EMB:2 * EMB, :], preferred_element_type=f32)
    su = su_ref[...]
    si = si_ref[...]
    pres = []
    for e in range(BLK):
        lo, hi = e * P, (e + 1) * P
        pre = (jnp.dot(su[lo:hi, :], gu[lo:hi, :], preferred_element_type=f32)
               + jnp.dot(si[lo:hi, :], gi[lo:hi, :], preferred_element_type=f32))
        pres.append(pre)
    pre1 = jnp.concatenate(pres, axis=0)
    h1 = jnp.maximum(pre1 + b1_ref[...], 0.0)
    h2 = jnp.maximum(
        jnp.dot(h1, w2_ref[...], preferred_element_type=f32) + b2_ref[...], 0.0)
    z = jnp.dot(h2, w3_ref[...], preferred_element_type=f32) + b3_ref[...]
    o = jax.nn.sigmoid(z)
    r = lax.broadcasted_iota(jnp.int32, (BLK * P, BLK), 0)
    c = lax.broadcasted_iota(jnp.int32, (BLK * P, BLK), 1)
    msk = ((r // P == c) & (r % P < NEIGH)).astype(f32)
    out_ref[0, 0, :] = jnp.sum(o * msk, axis=0) * (1.0 / NEIGH)


def _mlp_call(su, eu, si, ei, W1, b1, W2, b2, W3, b3):
    R = su.shape[0]
    B = R // P
    rows = BLK * P
    out = pl.pallas_call(
        _mlp_body,
        grid=(B // BLK,),
        in_specs=[
            pl.BlockSpec((rows, P), lambda i: (i, 0)),
            pl.BlockSpec((rows, EMB), lambda i: (i, 0)),
            pl.BlockSpec((rows, P), lambda i: (i, 0)),
            pl.BlockSpec((rows, EMB), lambda i: (i, 0)),
            pl.BlockSpec((2 * EMB, EMB), lambda i: (0, 0)),
            pl.BlockSpec((1, EMB), lambda i: (0, 0)),
            pl.BlockSpec((EMB, 8), lambda i: (0, 0)),
            pl.BlockSpec((1, 8), lambda i: (0, 0)),
            pl.BlockSpec((8, 1), lambda i: (0, 0)),
            pl.BlockSpec((1, 1), lambda i: (0, 0)),
        ],
        out_specs=pl.BlockSpec((1, 1, BLK), lambda i: (i, 0, 0)),
        out_shape=jax.ShapeDtypeStruct((B // BLK, 1, BLK), jnp.float32),
    )(su, eu, si, ei, W1, b1.reshape(1, EMB), W2, b2.reshape(1, 8),
      W3, b3.reshape(1, 1))
    return out.reshape(B)


def kernel(user_idxs, item_idxs, user_idx_tensor, item_idx_tensor,
           user_scr_tensor, item_scr_tensor, user_emb, item_emb,
           W1, b1, W2, b2, W3, b3):
    pad = ((0, 0), (0, P - NEIGH))
    utab = jnp.pad(user_idx_tensor, pad)
    itab = jnp.pad(item_idx_tensor, pad)
    uscr = jnp.pad(user_scr_tensor, pad)
    iscr = jnp.pad(item_scr_tensor, pad)
    su, eu, si, ei = _gather_all(user_idxs, item_idxs, utab, itab,
                                 uscr, iscr, user_emb, item_emb)
    return _mlp_call(su, eu, si, ei, W1, b1, W2, b2, W3, b3)


# speedup vs baseline: 2.6751x; 1.7124x over previous
import jax
import jax.numpy as jnp
from jax import lax
from jax.experimental import pallas as pl
from jax.experimental.pallas import tpu as pltpu
from jax.experimental.pallas import tpu_sc as plsc

NC, NS = 2, 16
NW = NC * NS
NEIGH = 50
P = 56
EMB = 16
G = 8
BLK = 8


def _mesh():
    return plsc.VectorSubcoreMesh(
        core_axis_name="c", subcore_axis_name="s", num_cores=NC, num_subcores=NS)


def _gather_all(user_idxs, item_idxs, utab, itab, uscr, iscr, uemb, iemb):
    B = user_idxs.shape[0]
    bpw = B // NW
    ngroups = bpw // G

    def body(uidx, iidx, utab_h, itab_h, uscr_h, iscr_h, uemb_h, iemb_h,
             su, eu, si, ei,
             idx_v, nu_v, ni_v, s_stg, e_stg, sem_n, sem_s, sem_e, sem_o):
        wid = lax.axis_index("s") * NC + lax.axis_index("c")
        ebase = wid * bpw

        pltpu.sync_copy(uidx.at[pl.ds(ebase, bpw)], idx_v.at[0])
        cu = pltpu.async_copy(utab_h.at[idx_v.at[0]], nu_v, sem_n)
        pltpu.sync_copy(iidx.at[pl.ds(ebase, bpw)], idx_v.at[1])
        ci = pltpu.async_copy(itab_h.at[idx_v.at[1]], ni_v, sem_n)
        cu.wait()
        ci.wait()

        for n_v, scr, emb, s_out, e_out in (
                (nu_v, uscr_h, uemb_h, su, eu), (ni_v, iscr_h, iemb_h, si, ei)):
            @pl.loop(0, ngroups)
            def _(g):
                p = lax.rem(g, 2)
                @pl.when(g >= 2)
                def _():
                    pltpu.make_async_copy(
                        s_stg.at[p], s_out.at[pl.ds(0, G * P)], sem_o).wait()
                    pltpu.make_async_copy(
                        e_stg.at[p], e_out.at[pl.ds(0, G * P)], sem_o).wait()
                for m in range(G):
                    e = g * G + m
                    pltpu.async_copy(
                        scr.at[n_v.at[e]], s_stg.at[p, pl.ds(m * P, P)], sem_s)
                    pltpu.async_copy(
                        emb.at[n_v.at[e]], e_stg.at[p, pl.ds(m * P, P)], sem_e)
                for m in range(G):
                    pltpu.make_async_copy(
                        scr.at[pl.ds(0, P)], s_stg.at[p, pl.ds(m * P, P)],
                        sem_s).wait()
                    pltpu.make_async_copy(
                        emb.at[pl.ds(0, P)], e_stg.at[p, pl.ds(m * P, P)],
                        sem_e).wait()
                rbase = (ebase + g * G) * P
                pltpu.async_copy(s_stg.at[p], s_out.at[pl.ds(rbase, G * P)],
                                 sem_o)
                pltpu.async_copy(e_stg.at[p], e_out.at[pl.ds(rbase, G * P)],
                                 sem_o)
            for _p in range(2):
                pltpu.make_async_copy(
                    s_stg.at[_p], s_out.at[pl.ds(0, G * P)], sem_o).wait()
                pltpu.make_async_copy(
                    e_stg.at[_p], e_out.at[pl.ds(0, G * P)], sem_o).wait()

    R = B * P
    call = pl.kernel(
        body,
        out_type=(
            jax.ShapeDtypeStruct((R, P), jnp.float32),
            jax.ShapeDtypeStruct((R, EMB), jnp.float32),
            jax.ShapeDtypeStruct((R, P), jnp.float32),
            jax.ShapeDtypeStruct((R, EMB), jnp.float32),
        ),
        mesh=_mesh(),
        scratch_types=[
            pltpu.VMEM((2, bpw), jnp.int32),
            pltpu.VMEM((bpw, P), jnp.int32),
            pltpu.VMEM((bpw, P), jnp.int32),
            pltpu.VMEM((2, G * P, P), jnp.float32),
            pltpu.VMEM((2, G * P, EMB), jnp.float32),
            pltpu.SemaphoreType.DMA,
            pltpu.SemaphoreType.DMA,
            pltpu.SemaphoreType.DMA,
            pltpu.SemaphoreType.DMA,
        ],
        compiler_params=pltpu.CompilerParams(use_tc_tiling_on_sc=False),
    )
    return call(user_idxs, item_idxs, utab, itab, uscr, iscr, uemb, iemb)


def _mlp_body(su_ref, eu_ref, si_ref, ei_ref, w1_ref, b1_ref, w2_ref, b2_ref,
              w3_ref, b3_ref, out_ref):
    f32 = jnp.float32
    w1 = w1_ref[...]
    gu = jnp.dot(eu_ref[...], w1[0:EMB, :], preferred_element_type=f32)
    gi = jnp.dot(ei_ref[...], w1[EMB:2 * EMB, :], preferred_element_type=f32)
    su = su_ref[...]
    si = si_ref[...]
    pres = []
    for e in range(BLK):
        lo, hi = e * P, (e + 1) * P
        pre = (jnp.dot(su[lo:hi, :], gu[lo:hi, :], preferred_element_type=f32)
               + jnp.dot(si[lo:hi, :], gi[lo:hi, :], preferred_element_type=f32))
        pres.append(pre)
    pre1 = jnp.concatenate(pres, axis=0)
    h1 = jnp.maximum(pre1 + b1_ref[...], 0.0)
    h2 = jnp.maximum(
        jnp.dot(h1, w2_ref[...], preferred_element_type=f32) + b2_ref[...], 0.0)
    z = jnp.dot(h2, w3_ref[...], preferred_element_type=f32) + b3_ref[...]
    o = jax.nn.sigmoid(z)
    r = lax.broadcasted_iota(jnp.int32, (BLK * P, BLK), 0)
    c = lax.broadcasted_iota(jnp.int32, (BLK * P, BLK), 1)
    msk = ((r // P == c) & (r % P < NEIGH)).astype(f32)
    out_ref[0, 0, :] = jnp.sum(o * msk, axis=0) * (1.0 / NEIGH)


def _mlp_call(su, eu, si, ei, W1, b1, W2, b2, W3, b3):
    R = su.shape[0]
    B = R // P
    rows = BLK * P
    out = pl.pallas_call(
        _mlp_body,
        grid=(B // BLK,),
        in_specs=[
            pl.BlockSpec((rows, P), lambda i: (i, 0)),
            pl.BlockSpec((rows, EMB), lambda i: (i, 0)),
            pl.BlockSpec((rows, P), lambda i: (i, 0)),
            pl.BlockSpec((rows, EMB), lambda i: (i, 0)),
            pl.BlockSpec((2 * EMB, EMB), lambda i: (0, 0)),
            pl.BlockSpec((1, EMB), lambda i: (0, 0)),
            pl.BlockSpec((EMB, 8), lambda i: (0, 0)),
            pl.BlockSpec((1, 8), lambda i: (0, 0)),
            pl.BlockSpec((8, 1), lambda i: (0, 0)),
            pl.BlockSpec((1, 1), lambda i: (0, 0)),
        ],
        out_specs=pl.BlockSpec((1, 1, BLK), lambda i: (i, 0, 0)),
        out_shape=jax.ShapeDtypeStruct((B // BLK, 1, BLK), jnp.float32),
    )(su, eu, si, ei, W1, b1.reshape(1, EMB), W2, b2.reshape(1, 8),
      W3, b3.reshape(1, 1))
    return out.reshape(B)


def kernel(user_idxs, item_idxs, user_idx_tensor, item_idx_tensor,
           user_scr_tensor, item_scr_tensor, user_emb, item_emb,
           W1, b1, W2, b2, W3, b3):
    pad = ((0, 0), (0, P - NEIGH))
    utab = jnp.concatenate(
        [user_idx_tensor, user_idx_tensor[:, :P - NEIGH]], axis=1)
    itab = jnp.concatenate(
        [item_idx_tensor, item_idx_tensor[:, :P - NEIGH]], axis=1)
    uscr = jnp.pad(user_scr_tensor, pad)
    iscr = jnp.pad(item_scr_tensor, pad)
    su, eu, si, ei = _gather_all(user_idxs, item_idxs, utab, itab,
                                 uscr, iscr, user_emb, item_emb)
    return _mlp_call(su, eu, si, ei, W1, b1, W2, b2, W3, b3)


# speedup vs baseline: 3.0562x; 1.1425x over previous
import jax
import jax.numpy as jnp
from jax import lax
from jax.experimental import pallas as pl
from jax.experimental.pallas import tpu as pltpu
from jax.experimental.pallas import tpu_sc as plsc

NC, NS = 2, 16
NW = NC * NS
NEIGH = 50
P = 56
EMB = 16
G = 8
BLK = 16


def _mesh():
    return plsc.VectorSubcoreMesh(
        core_axis_name="c", subcore_axis_name="s", num_cores=NC, num_subcores=NS)


def _gather_all(user_idxs, item_idxs, utab, itab, uscr, iscr, uemb, iemb):
    B = user_idxs.shape[0]
    bpw = B // NW
    ngroups = bpw // G

    def body(uidx, iidx, utab_h, itab_h, uscr_h, iscr_h, uemb_h, iemb_h,
             su, eu, si, ei,
             idx_v, nu_v, ni_v, s_stg, e_stg, sem_n, sem_s, sem_e, sem_o):
        wid = lax.axis_index("s") * NC + lax.axis_index("c")
        ebase = wid * bpw

        pltpu.sync_copy(uidx.at[pl.ds(ebase, bpw)], idx_v.at[0])
        cu = pltpu.async_copy(utab_h.at[idx_v.at[0]], nu_v, sem_n)
        pltpu.sync_copy(iidx.at[pl.ds(ebase, bpw)], idx_v.at[1])
        ci = pltpu.async_copy(itab_h.at[idx_v.at[1]], ni_v, sem_n)
        cu.wait()
        ci.wait()

        for n_v, scr, emb, s_out, e_out in (
                (nu_v, uscr_h, uemb_h, su, eu), (ni_v, iscr_h, iemb_h, si, ei)):
            @pl.loop(0, ngroups)
            def _(g):
                p = lax.rem(g, 2)
                @pl.when(g >= 2)
                def _():
                    pltpu.make_async_copy(
                        s_stg.at[p], s_out.at[pl.ds(0, G * P)], sem_o).wait()
                    pltpu.make_async_copy(
                        e_stg.at[p], e_out.at[pl.ds(0, G * P)], sem_o).wait()
                for m in range(G):
                    e = g * G + m
                    pltpu.async_copy(
                        scr.at[n_v.at[e]], s_stg.at[p, pl.ds(m * P, P)], sem_s)
                    pltpu.async_copy(
                        emb.at[n_v.at[e]], e_stg.at[p, pl.ds(m * P, P)], sem_e)
                for m in range(G):
                    pltpu.make_async_copy(
                        scr.at[pl.ds(0, P)], s_stg.at[p, pl.ds(m * P, P)],
                        sem_s).wait()
                    pltpu.make_async_copy(
                        emb.at[pl.ds(0, P)], e_stg.at[p, pl.ds(m * P, P)],
                        sem_e).wait()
                rbase = (ebase + g * G) * P
                pltpu.async_copy(s_stg.at[p], s_out.at[pl.ds(rbase, G * P)],
                                 sem_o)
                pltpu.async_copy(e_stg.at[p], e_out.at[pl.ds(rbase, G * P)],
                                 sem_o)
            for _p in range(2):
                pltpu.make_async_copy(
                    s_stg.at[_p], s_out.at[pl.ds(0, G * P)], sem_o).wait()
                pltpu.make_async_copy(
                    e_stg.at[_p], e_out.at[pl.ds(0, G * P)], sem_o).wait()

    R = B * P
    call = pl.kernel(
        body,
        out_type=(
            jax.ShapeDtypeStruct((R, P), jnp.float32),
            jax.ShapeDtypeStruct((R, EMB), jnp.float32),
            jax.ShapeDtypeStruct((R, P), jnp.float32),
            jax.ShapeDtypeStruct((R, EMB), jnp.float32),
        ),
        mesh=_mesh(),
        scratch_types=[
            pltpu.VMEM((2, bpw), jnp.int32),
            pltpu.VMEM((bpw, P), jnp.int32),
            pltpu.VMEM((bpw, P), jnp.int32),
            pltpu.VMEM((2, G * P, P), jnp.float32),
            pltpu.VMEM((2, G * P, EMB), jnp.float32),
            pltpu.SemaphoreType.DMA,
            pltpu.SemaphoreType.DMA,
            pltpu.SemaphoreType.DMA,
            pltpu.SemaphoreType.DMA,
        ],
        compiler_params=pltpu.CompilerParams(use_tc_tiling_on_sc=False),
    )
    return call(user_idxs, item_idxs, utab, itab, uscr, iscr, uemb, iemb)


def _mlp_body(su_ref, eu_ref, si_ref, ei_ref, w1_ref, b1_ref, w2_ref, b2_ref,
              w3_ref, b3_ref, out_ref):
    f32 = jnp.float32
    w1 = w1_ref[...]
    gu = jnp.dot(eu_ref[...], w1[0:EMB, :], preferred_element_type=f32)
    gi = jnp.dot(ei_ref[...], w1[EMB:2 * EMB, :], preferred_element_type=f32)
    su = su_ref[...]
    si = si_ref[...]
    pres = []
    for e in range(BLK):
        lo, hi = e * P, (e + 1) * P
        pre = (jnp.dot(su[lo:hi, :], gu[lo:hi, :], preferred_element_type=f32)
               + jnp.dot(si[lo:hi, :], gi[lo:hi, :], preferred_element_type=f32))
        pres.append(pre)
    pre1 = jnp.concatenate(pres, axis=0)
    h1 = jnp.maximum(pre1 + b1_ref[...], 0.0)
    h2 = jnp.maximum(
        jnp.dot(h1, w2_ref[...], preferred_element_type=f32) + b2_ref[...], 0.0)
    z = jnp.dot(h2, w3_ref[...], preferred_element_type=f32) + b3_ref[...]
    o = jax.nn.sigmoid(z)
    r = lax.broadcasted_iota(jnp.int32, (BLK * P, BLK), 0)
    c = lax.broadcasted_iota(jnp.int32, (BLK * P, BLK), 1)
    msk = ((r // P == c) & (r % P < NEIGH)).astype(f32)
    out_ref[0, 0, :] = jnp.sum(o * msk, axis=0) * (1.0 / NEIGH)


def _mlp_call(su, eu, si, ei, W1, b1, W2, b2, W3, b3):
    R = su.shape[0]
    B = R // P
    rows = BLK * P
    out = pl.pallas_call(
        _mlp_body,
        grid=(B // BLK,),
        in_specs=[
            pl.BlockSpec((rows, P), lambda i: (i, 0)),
            pl.BlockSpec((rows, EMB), lambda i: (i, 0)),
            pl.BlockSpec((rows, P), lambda i: (i, 0)),
            pl.BlockSpec((rows, EMB), lambda i: (i, 0)),
            pl.BlockSpec((2 * EMB, EMB), lambda i: (0, 0)),
            pl.BlockSpec((1, EMB), lambda i: (0, 0)),
            pl.BlockSpec((EMB, 8), lambda i: (0, 0)),
            pl.BlockSpec((1, 8), lambda i: (0, 0)),
            pl.BlockSpec((8, 1), lambda i: (0, 0)),
            pl.BlockSpec((1, 1), lambda i: (0, 0)),
        ],
        out_specs=pl.BlockSpec((1, 1, BLK), lambda i: (i, 0, 0)),
        out_shape=jax.ShapeDtypeStruct((B // BLK, 1, BLK), jnp.float32),
    )(su, eu, si, ei, W1, b1.reshape(1, EMB), W2, b2.reshape(1, 8),
      W3, b3.reshape(1, 1))
    return out.reshape(B)


def kernel(user_idxs, item_idxs, user_idx_tensor, item_idx_tensor,
           user_scr_tensor, item_scr_tensor, user_emb, item_emb,
           W1, b1, W2, b2, W3, b3):
    pad = ((0, 0), (0, P - NEIGH))
    utab = jnp.concatenate(
        [user_idx_tensor, user_idx_tensor[:, :P - NEIGH]], axis=1)
    itab = jnp.concatenate(
        [item_idx_tensor, item_idx_tensor[:, :P - NEIGH]], axis=1)
    uscr = jnp.pad(user_scr_tensor, pad)
    iscr = jnp.pad(item_scr_tensor, pad)
    su, eu, si, ei = _gather_all(user_idxs, item_idxs, utab, itab,
                                 uscr, iscr, user_emb, item_emb)
    return _mlp_call(su, eu, si, ei, W1, b1, W2, b2, W3, b3)


# speedup vs baseline: 3.2568x; 1.0656x over previous
import jax
import jax.numpy as jnp
from jax import lax
from jax.experimental import pallas as pl
from jax.experimental.pallas import tpu as pltpu
from jax.experimental.pallas import tpu_sc as plsc

NC, NS = 2, 16
NW = NC * NS
NEIGH = 50
P = 56
EMB = 16
G = 8
BLK = 32


def _mesh():
    return plsc.VectorSubcoreMesh(
        core_axis_name="c", subcore_axis_name="s", num_cores=NC, num_subcores=NS)


def _gather_all(user_idxs, item_idxs, utab, itab, uscr, iscr, uemb, iemb):
    B = user_idxs.shape[0]
    bpw = B // NW
    ngroups = bpw // G

    def body(uidx, iidx, utab_h, itab_h, uscr_h, iscr_h, uemb_h, iemb_h,
             su, eu, si, ei,
             idx_v, nu_v, ni_v, s_stg, e_stg, sem_n, sem_s, sem_e, sem_o):
        wid = lax.axis_index("s") * NC + lax.axis_index("c")
        ebase = wid * bpw

        pltpu.sync_copy(uidx.at[pl.ds(ebase, bpw)], idx_v.at[0])
        cu = pltpu.async_copy(utab_h.at[idx_v.at[0]], nu_v, sem_n)
        pltpu.sync_copy(iidx.at[pl.ds(ebase, bpw)], idx_v.at[1])
        ci = pltpu.async_copy(itab_h.at[idx_v.at[1]], ni_v, sem_n)
        cu.wait()
        ci.wait()

        for n_v, scr, emb, s_out, e_out in (
                (nu_v, uscr_h, uemb_h, su, eu), (ni_v, iscr_h, iemb_h, si, ei)):
            @pl.loop(0, ngroups)
            def _(g):
                p = lax.rem(g, 2)
                @pl.when(g >= 2)
                def _():
                    pltpu.make_async_copy(
                        s_stg.at[p], s_out.at[pl.ds(0, G * P)], sem_o).wait()
                    pltpu.make_async_copy(
                        e_stg.at[p], e_out.at[pl.ds(0, G * P)], sem_o).wait()
                for m in range(G):
                    e = g * G + m
                    pltpu.async_copy(
                        scr.at[n_v.at[e]], s_stg.at[p, pl.ds(m * P, P)], sem_s)
                    pltpu.async_copy(
                        emb.at[n_v.at[e]], e_stg.at[p, pl.ds(m * P, P)], sem_e)
                for m in range(G):
                    pltpu.make_async_copy(
                        scr.at[pl.ds(0, P)], s_stg.at[p, pl.ds(m * P, P)],
                        sem_s).wait()
                    pltpu.make_async_copy(
                        emb.at[pl.ds(0, P)], e_stg.at[p, pl.ds(m * P, P)],
                        sem_e).wait()
                rbase = (ebase + g * G) * P
                pltpu.async_copy(s_stg.at[p], s_out.at[pl.ds(rbase, G * P)],
                                 sem_o)
                pltpu.async_copy(e_stg.at[p], e_out.at[pl.ds(rbase, G * P)],
                                 sem_o)
            for _p in range(2):
                pltpu.make_async_copy(
                    s_stg.at[_p], s_out.at[pl.ds(0, G * P)], sem_o).wait()
                pltpu.make_async_copy(
                    e_stg.at[_p], e_out.at[pl.ds(0, G * P)], sem_o).wait()

    R = B * P
    call = pl.kernel(
        body,
        out_type=(
            jax.ShapeDtypeStruct((R, P), jnp.float32),
            jax.ShapeDtypeStruct((R, EMB), jnp.float32),
            jax.ShapeDtypeStruct((R, P), jnp.float32),
            jax.ShapeDtypeStruct((R, EMB), jnp.float32),
        ),
        mesh=_mesh(),
        scratch_types=[
            pltpu.VMEM((2, bpw), jnp.int32),
            pltpu.VMEM((bpw, P), jnp.int32),
            pltpu.VMEM((bpw, P), jnp.int32),
            pltpu.VMEM((2, G * P, P), jnp.float32),
            pltpu.VMEM((2, G * P, EMB), jnp.float32),
            pltpu.SemaphoreType.DMA,
            pltpu.SemaphoreType.DMA,
            pltpu.SemaphoreType.DMA,
            pltpu.SemaphoreType.DMA,
        ],
        compiler_params=pltpu.CompilerParams(use_tc_tiling_on_sc=False),
    )
    return call(user_idxs, item_idxs, utab, itab, uscr, iscr, uemb, iemb)


def _mlp_body(su_ref, eu_ref, si_ref, ei_ref, w1_ref, b1_ref, w2_ref, b2_ref,
              w3_ref, b3_ref, out_ref):
    f32 = jnp.float32
    w1 = w1_ref[...]
    gu = jnp.dot(eu_ref[...], w1[0:EMB, :], preferred_element_type=f32)
    gi = jnp.dot(ei_ref[...], w1[EMB:2 * EMB, :], preferred_element_type=f32)
    su = su_ref[...]
    si = si_ref[...]
    pres = []
    for e in range(BLK):
        lo, hi = e * P, (e + 1) * P
        pre = (jnp.dot(su[lo:hi, :], gu[lo:hi, :], preferred_element_type=f32)
               + jnp.dot(si[lo:hi, :], gi[lo:hi, :], preferred_element_type=f32))
        pres.append(pre)
    pre1 = jnp.concatenate(pres, axis=0)
    h1 = jnp.maximum(pre1 + b1_ref[...], 0.0)
    h2 = jnp.maximum(
        jnp.dot(h1, w2_ref[...], preferred_element_type=f32) + b2_ref[...], 0.0)
    z = jnp.dot(h2, w3_ref[...], preferred_element_type=f32) + b3_ref[...]
    o = jax.nn.sigmoid(z)
    r = lax.broadcasted_iota(jnp.int32, (BLK * P, BLK), 0)
    c = lax.broadcasted_iota(jnp.int32, (BLK * P, BLK), 1)
    msk = ((r // P == c) & (r % P < NEIGH)).astype(f32)
    out_ref[0, 0, :] = jnp.sum(o * msk, axis=0) * (1.0 / NEIGH)


def _mlp_call(su, eu, si, ei, W1, b1, W2, b2, W3, b3):
    R = su.shape[0]
    B = R // P
    rows = BLK * P
    out = pl.pallas_call(
        _mlp_body,
        grid=(B // BLK,),
        in_specs=[
            pl.BlockSpec((rows, P), lambda i: (i, 0)),
            pl.BlockSpec((rows, EMB), lambda i: (i, 0)),
            pl.BlockSpec((rows, P), lambda i: (i, 0)),
            pl.BlockSpec((rows, EMB), lambda i: (i, 0)),
            pl.BlockSpec((2 * EMB, EMB), lambda i: (0, 0)),
            pl.BlockSpec((1, EMB), lambda i: (0, 0)),
            pl.BlockSpec((EMB, 8), lambda i: (0, 0)),
            pl.BlockSpec((1, 8), lambda i: (0, 0)),
            pl.BlockSpec((8, 1), lambda i: (0, 0)),
            pl.BlockSpec((1, 1), lambda i: (0, 0)),
        ],
        out_specs=pl.BlockSpec((1, 1, BLK), lambda i: (i, 0, 0)),
        out_shape=jax.ShapeDtypeStruct((B // BLK, 1, BLK), jnp.float32),
    )(su, eu, si, ei, W1, b1.reshape(1, EMB), W2, b2.reshape(1, 8),
      W3, b3.reshape(1, 1))
    return out.reshape(B)


def kernel(user_idxs, item_idxs, user_idx_tensor, item_idx_tensor,
           user_scr_tensor, item_scr_tensor, user_emb, item_emb,
           W1, b1, W2, b2, W3, b3):
    pad = ((0, 0), (0, P - NEIGH))
    utab = jnp.concatenate(
        [user_idx_tensor, user_idx_tensor[:, :P - NEIGH]], axis=1)
    itab = jnp.concatenate(
        [item_idx_tensor, item_idx_tensor[:, :P - NEIGH]], axis=1)
    uscr = jnp.pad(user_scr_tensor, pad)
    iscr = jnp.pad(item_scr_tensor, pad)
    su, eu, si, ei = _gather_all(user_idxs, item_idxs, utab, itab,
                                 uscr, iscr, user_emb, item_emb)
    return _mlp_call(su, eu, si, ei, W1, b1, W2, b2, W3, b3)


# speedup vs baseline: 3.3630x; 1.0326x over previous
import jax
import jax.numpy as jnp
from jax import lax
from jax.experimental import pallas as pl
from jax.experimental.pallas import tpu as pltpu
from jax.experimental.pallas import tpu_sc as plsc

NC, NS = 2, 16
NW = NC * NS
NEIGH = 50
P = 56
EMB = 16
G = 8
BLK = 64


def _mesh():
    return plsc.VectorSubcoreMesh(
        core_axis_name="c", subcore_axis_name="s", num_cores=NC, num_subcores=NS)


def _gather_all(user_idxs, item_idxs, utab, itab, uscr, iscr, uemb, iemb):
    B = user_idxs.shape[0]
    bpw = B // NW
    ngroups = bpw // G

    def body(uidx, iidx, utab_h, itab_h, uscr_h, iscr_h, uemb_h, iemb_h,
             su, eu, si, ei,
             idx_v, nu_v, ni_v, s_stg, e_stg, sem_n, sem_s, sem_e, sem_o):
        wid = lax.axis_index("s") * NC + lax.axis_index("c")
        ebase = wid * bpw

        pltpu.sync_copy(uidx.at[pl.ds(ebase, bpw)], idx_v.at[0])
        cu = pltpu.async_copy(utab_h.at[idx_v.at[0]], nu_v, sem_n)
        pltpu.sync_copy(iidx.at[pl.ds(ebase, bpw)], idx_v.at[1])
        ci = pltpu.async_copy(itab_h.at[idx_v.at[1]], ni_v, sem_n)
        cu.wait()
        ci.wait()

        for n_v, scr, emb, s_out, e_out in (
                (nu_v, uscr_h, uemb_h, su, eu), (ni_v, iscr_h, iemb_h, si, ei)):
            @pl.loop(0, ngroups)
            def _(g):
                p = lax.rem(g, 2)
                @pl.when(g >= 2)
                def _():
                    pltpu.make_async_copy(
                        s_stg.at[p], s_out.at[pl.ds(0, G * P)], sem_o).wait()
                    pltpu.make_async_copy(
                        e_stg.at[p], e_out.at[pl.ds(0, G * P)], sem_o).wait()
                for m in range(G):
                    e = g * G + m
                    pltpu.async_copy(
                        scr.at[n_v.at[e]], s_stg.at[p, pl.ds(m * P, P)], sem_s)
                    pltpu.async_copy(
                        emb.at[n_v.at[e]], e_stg.at[p, pl.ds(m * P, P)], sem_e)
                for m in range(G):
                    pltpu.make_async_copy(
                        scr.at[pl.ds(0, P)], s_stg.at[p, pl.ds(m * P, P)],
                        sem_s).wait()
                    pltpu.make_async_copy(
                        emb.at[pl.ds(0, P)], e_stg.at[p, pl.ds(m * P, P)],
                        sem_e).wait()
                rbase = (ebase + g * G) * P
                pltpu.async_copy(s_stg.at[p], s_out.at[pl.ds(rbase, G * P)],
                                 sem_o)
                pltpu.async_copy(e_stg.at[p], e_out.at[pl.ds(rbase, G * P)],
                                 sem_o)
            for _p in range(2):
                pltpu.make_async_copy(
                    s_stg.at[_p], s_out.at[pl.ds(0, G * P)], sem_o).wait()
                pltpu.make_async_copy(
                    e_stg.at[_p], e_out.at[pl.ds(0, G * P)], sem_o).wait()

    R = B * P
    call = pl.kernel(
        body,
        out_type=(
            jax.ShapeDtypeStruct((R, P), jnp.float32),
            jax.ShapeDtypeStruct((R, EMB), jnp.float32),
            jax.ShapeDtypeStruct((R, P), jnp.float32),
            jax.ShapeDtypeStruct((R, EMB), jnp.float32),
        ),
        mesh=_mesh(),
        scratch_types=[
            pltpu.VMEM((2, bpw), jnp.int32),
            pltpu.VMEM((bpw, P), jnp.int32),
            pltpu.VMEM((bpw, P), jnp.int32),
            pltpu.VMEM((2, G * P, P), jnp.float32),
            pltpu.VMEM((2, G * P, EMB), jnp.float32),
            pltpu.SemaphoreType.DMA,
            pltpu.SemaphoreType.DMA,
            pltpu.SemaphoreType.DMA,
            pltpu.SemaphoreType.DMA,
        ],
        compiler_params=pltpu.CompilerParams(use_tc_tiling_on_sc=False),
    )
    return call(user_idxs, item_idxs, utab, itab, uscr, iscr, uemb, iemb)


def _mlp_body(su_ref, eu_ref, si_ref, ei_ref, w1_ref, b1_ref, w2_ref, b2_ref,
              w3_ref, b3_ref, out_ref):
    f32 = jnp.float32
    w1 = w1_ref[...]
    gu = jnp.dot(eu_ref[...], w1[0:EMB, :], preferred_element_type=f32)
    gi = jnp.dot(ei_ref[...], w1[EMB:2 * EMB, :], preferred_element_type=f32)
    su = su_ref[...]
    si = si_ref[...]
    pres = []
    for e in range(BLK):
        lo, hi = e * P, (e + 1) * P
        pre = (jnp.dot(su[lo:hi, :], gu[lo:hi, :], preferred_element_type=f32)
               + jnp.dot(si[lo:hi, :], gi[lo:hi, :], preferred_element_type=f32))
        pres.append(pre)
    pre1 = jnp.concatenate(pres, axis=0)
    h1 = jnp.maximum(pre1 + b1_ref[...], 0.0)
    h2 = jnp.maximum(
        jnp.dot(h1, w2_ref[...], preferred_element_type=f32) + b2_ref[...], 0.0)
    z = jnp.dot(h2, w3_ref[...], preferred_element_type=f32) + b3_ref[...]
    o = jax.nn.sigmoid(z)
    r = lax.broadcasted_iota(jnp.int32, (BLK * P, BLK), 0)
    c = lax.broadcasted_iota(jnp.int32, (BLK * P, BLK), 1)
    msk = ((r // P == c) & (r % P < NEIGH)).astype(f32)
    out_ref[0, 0, :] = jnp.sum(o * msk, axis=0) * (1.0 / NEIGH)


def _mlp_call(su, eu, si, ei, W1, b1, W2, b2, W3, b3):
    R = su.shape[0]
    B = R // P
    rows = BLK * P
    out = pl.pallas_call(
        _mlp_body,
        grid=(B // BLK,),
        in_specs=[
            pl.BlockSpec((rows, P), lambda i: (i, 0)),
            pl.BlockSpec((rows, EMB), lambda i: (i, 0)),
            pl.BlockSpec((rows, P), lambda i: (i, 0)),
            pl.BlockSpec((rows, EMB), lambda i: (i, 0)),
            pl.BlockSpec((2 * EMB, EMB), lambda i: (0, 0)),
            pl.BlockSpec((1, EMB), lambda i: (0, 0)),
            pl.BlockSpec((EMB, 8), lambda i: (0, 0)),
            pl.BlockSpec((1, 8), lambda i: (0, 0)),
            pl.BlockSpec((8, 1), lambda i: (0, 0)),
            pl.BlockSpec((1, 1), lambda i: (0, 0)),
        ],
        out_specs=pl.BlockSpec((1, 1, BLK), lambda i: (i, 0, 0)),
        out_shape=jax.ShapeDtypeStruct((B // BLK, 1, BLK), jnp.float32),
    )(su, eu, si, ei, W1, b1.reshape(1, EMB), W2, b2.reshape(1, 8),
      W3, b3.reshape(1, 1))
    return out.reshape(B)


def kernel(user_idxs, item_idxs, user_idx_tensor, item_idx_tensor,
           user_scr_tensor, item_scr_tensor, user_emb, item_emb,
           W1, b1, W2, b2, W3, b3):
    pad = ((0, 0), (0, P - NEIGH))
    utab = jnp.concatenate(
        [user_idx_tensor, user_idx_tensor[:, :P - NEIGH]], axis=1)
    itab = jnp.concatenate(
        [item_idx_tensor, item_idx_tensor[:, :P - NEIGH]], axis=1)
    uscr = jnp.pad(user_scr_tensor, pad)
    iscr = jnp.pad(item_scr_tensor, pad)
    su, eu, si, ei = _gather_all(user_idxs, item_idxs, utab, itab,
                                 uscr, iscr, user_emb, item_emb)
    return _mlp_call(su, eu, si, ei, W1, b1, W2, b2, W3, b3)


# speedup vs baseline: 3.6891x; 1.0970x over previous
import jax
import jax.numpy as jnp
from jax import lax
from jax.experimental import pallas as pl
from jax.experimental.pallas import tpu as pltpu
from jax.experimental.pallas import tpu_sc as plsc

NC, NS = 2, 16
NW = NC * NS
NEIGH = 50
P = 56
W = 64
HR = P * W // 128
EMB = 16
G = 8
BLK = 64


def _mesh():
    return plsc.VectorSubcoreMesh(
        core_axis_name="c", subcore_axis_name="s", num_cores=NC, num_subcores=NS)


def _gather_all(user_idxs, item_idxs, utab, itab, uscr, iscr, uemb, iemb):
    B = user_idxs.shape[0]
    bpw = B // NW
    ngroups = bpw // G

    def body(uidx, iidx, utab_h, itab_h, uscr_h, iscr_h, uemb_h, iemb_h,
             su, eu, si, ei,
             idx_v, nu_v, ni_v, s_stg, e_stg, sem_n, sem_s, sem_e, sem_o):
        wid = lax.axis_index("s") * NC + lax.axis_index("c")
        ebase = wid * bpw

        pltpu.sync_copy(uidx.at[pl.ds(ebase, bpw)], idx_v.at[0])
        cu = pltpu.async_copy(utab_h.at[idx_v.at[0]], nu_v, sem_n)
        pltpu.sync_copy(iidx.at[pl.ds(ebase, bpw)], idx_v.at[1])
        ci = pltpu.async_copy(itab_h.at[idx_v.at[1]], ni_v, sem_n)
        cu.wait()
        ci.wait()

        for n_v, scr, emb, s_out, e_out in (
                (nu_v, uscr_h, uemb_h, su, eu), (ni_v, iscr_h, iemb_h, si, ei)):
            @pl.loop(0, ngroups)
            def _(g):
                p = lax.rem(g, 2)
                @pl.when(g >= 2)
                def _():
                    pltpu.make_async_copy(
                        s_stg.at[p], s_out.at[pl.ds(0, G * P)], sem_o).wait()
                    pltpu.make_async_copy(
                        e_stg.at[p], e_out.at[pl.ds(0, G * P)], sem_o).wait()
                for m in range(G):
                    e = g * G + m
                    pltpu.async_copy(
                        scr.at[n_v.at[e]], s_stg.at[p, pl.ds(m * P, P)], sem_s)
                    pltpu.async_copy(
                        emb.at[n_v.at[e]], e_stg.at[p, pl.ds(m * P, P)], sem_e)
                for m in range(G):
                    pltpu.make_async_copy(
                        scr.at[pl.ds(0, P)], s_stg.at[p, pl.ds(m * P, P)],
                        sem_s).wait()
                    pltpu.make_async_copy(
                        emb.at[pl.ds(0, P)], e_stg.at[p, pl.ds(m * P, P)],
                        sem_e).wait()
                rbase = (ebase + g * G) * P
                pltpu.async_copy(s_stg.at[p], s_out.at[pl.ds(rbase, G * P)],
                                 sem_o)
                pltpu.async_copy(e_stg.at[p], e_out.at[pl.ds(rbase, G * P)],
                                 sem_o)
            for _p in range(2):
                pltpu.make_async_copy(
                    s_stg.at[_p], s_out.at[pl.ds(0, G * P)], sem_o).wait()
                pltpu.make_async_copy(
                    e_stg.at[_p], e_out.at[pl.ds(0, G * P)], sem_o).wait()

    R = B * P
    call = pl.kernel(
        body,
        out_type=(
            jax.ShapeDtypeStruct((R, W), jnp.float32),
            jax.ShapeDtypeStruct((R, EMB), jnp.float32),
            jax.ShapeDtypeStruct((R, W), jnp.float32),
            jax.ShapeDtypeStruct((R, EMB), jnp.float32),
        ),
        mesh=_mesh(),
        scratch_types=[
            pltpu.VMEM((2, bpw), jnp.int32),
            pltpu.VMEM((bpw, P), jnp.int32),
            pltpu.VMEM((bpw, P), jnp.int32),
            pltpu.VMEM((2, G * P, W), jnp.float32),
            pltpu.VMEM((2, G * P, EMB), jnp.float32),
            pltpu.SemaphoreType.DMA,
            pltpu.SemaphoreType.DMA,
            pltpu.SemaphoreType.DMA,
            pltpu.SemaphoreType.DMA,
        ],
        compiler_params=pltpu.CompilerParams(use_tc_tiling_on_sc=False),
    )
    return call(user_idxs, item_idxs, utab, itab, uscr, iscr, uemb, iemb)


def _mlp_body(su_ref, eu_ref, si_ref, ei_ref, w1_ref, b1_ref, w2_ref, b2_ref,
              w3_ref, b3_ref, out_ref):
    f32 = jnp.float32
    w1 = w1_ref[...]
    gu = jnp.dot(eu_ref[...], w1[0:EMB, :], preferred_element_type=f32)
    gi = jnp.dot(ei_ref[...], w1[EMB:2 * EMB, :], preferred_element_type=f32)
    su = su_ref[...]
    si = si_ref[...]
    z72 = jnp.zeros((2 * W - P, EMB), f32)
    z64 = jnp.zeros((W, EMB), f32)
    z8 = jnp.zeros((W - P, EMB), f32)
    evens, odds = [], []
    for e in range(BLK):
        fsu = su[e * HR:(e + 1) * HR, :]
        fsi = si[e * HR:(e + 1) * HR, :]
        gue = gu[e * P:(e + 1) * P, :]
        gie = gi[e * P:(e + 1) * P, :]
        gtop_u = jnp.concatenate([gue, z72], axis=0)
        gbot_u = jnp.concatenate([z64, gue, z8], axis=0)
        gtop_i = jnp.concatenate([gie, z72], axis=0)
        gbot_i = jnp.concatenate([z64, gie, z8], axis=0)
        evens.append(jnp.dot(fsu, gtop_u, preferred_element_type=f32)
                     + jnp.dot(fsi, gtop_i, preferred_element_type=f32))
        odds.append(jnp.dot(fsu, gbot_u, preferred_element_type=f32)
                    + jnp.dot(fsi, gbot_i, preferred_element_type=f32))
    pre1 = jnp.concatenate(evens + odds, axis=0)
    h1 = jnp.maximum(pre1 + b1_ref[...], 0.0)
    h2 = jnp.maximum(
        jnp.dot(h1, w2_ref[...], preferred_element_type=f32) + b2_ref[...], 0.0)
    z = jnp.dot(h2, w3_ref[...], preferred_element_type=f32) + b3_ref[...]
    o = jax.nn.sigmoid(z)
    half = BLK * HR
    t = lax.broadcasted_iota(jnp.int32, (2 * half, BLK), 0)
    c = lax.broadcasted_iota(jnp.int32, (2 * half, BLK), 1)
    th = t % half
    msk = ((th // HR == c) & (th % HR < (NEIGH + 1) // 2)).astype(f32)
    out_ref[0, 0, :] = jnp.sum(o * msk, axis=0) * (1.0 / NEIGH)


def _mlp_call(su, eu, si, ei, W1, b1, W2, b2, W3, b3):
    B = eu.shape[0] // P
    srows = BLK * HR
    erows = BLK * P
    out = pl.pallas_call(
        _mlp_body,
        grid=(B // BLK,),
        in_specs=[
            pl.BlockSpec((srows, 128), lambda i: (i, 0)),
            pl.BlockSpec((erows, EMB), lambda i: (i, 0)),
            pl.BlockSpec((srows, 128), lambda i: (i, 0)),
            pl.BlockSpec((erows, EMB), lambda i: (i, 0)),
            pl.BlockSpec((2 * EMB, EMB), lambda i: (0, 0)),
            pl.BlockSpec((1, EMB), lambda i: (0, 0)),
            pl.BlockSpec((EMB, 8), lambda i: (0, 0)),
            pl.BlockSpec((1, 8), lambda i: (0, 0)),
            pl.BlockSpec((8, 1), lambda i: (0, 0)),
            pl.BlockSpec((1, 1), lambda i: (0, 0)),
        ],
        out_specs=pl.BlockSpec((1, 1, BLK), lambda i: (i, 0, 0)),
        out_shape=jax.ShapeDtypeStruct((B // BLK, 1, BLK), jnp.float32),
    )(su, eu, si, ei, W1, b1.reshape(1, EMB), W2, b2.reshape(1, 8),
      W3, b3.reshape(1, 1))
    return out.reshape(B)


def kernel(user_idxs, item_idxs, user_idx_tensor, item_idx_tensor,
           user_scr_tensor, item_scr_tensor, user_emb, item_emb,
           W1, b1, W2, b2, W3, b3):
    utab = jnp.concatenate(
        [user_idx_tensor, user_idx_tensor[:, :P - NEIGH]], axis=1)
    itab = jnp.concatenate(
        [item_idx_tensor, item_idx_tensor[:, :P - NEIGH]], axis=1)
    padw = ((0, 0), (0, W - NEIGH))
    uscr = jnp.pad(user_scr_tensor, padw)
    iscr = jnp.pad(item_scr_tensor, padw)
    su, eu, si, ei = _gather_all(user_idxs, item_idxs, utab, itab,
                                 uscr, iscr, user_emb, item_emb)
    su = su.reshape(-1, 128)
    si = si.reshape(-1, 128)
    return _mlp_call(su, eu, si, ei, W1, b1, W2, b2, W3, b3)


# speedup vs baseline: 3.8335x; 1.0392x over previous
import jax
import jax.numpy as jnp
from jax import lax
from jax.experimental import pallas as pl
from jax.experimental.pallas import tpu as pltpu
from jax.experimental.pallas import tpu_sc as plsc

NC, NS = 2, 16
NW = NC * NS
NEIGH = 50
P = 56
W = 64
HR = P * W // 128
EMB = 16
G = 8
BLK = 64


def _mesh():
    return plsc.VectorSubcoreMesh(
        core_axis_name="c", subcore_axis_name="s", num_cores=NC, num_subcores=NS)


def _gather_all(user_idxs, item_idxs, utab, itab, uscr, iscr, uemb, iemb):
    B = user_idxs.shape[0]
    bpw = B // NW
    ngroups = bpw // G

    def body(uidx, iidx, utab_h, itab_h, uscr_h, iscr_h, uemb_h, iemb_h,
             su, eu, si, ei,
             idx_v, nu_v, ni_v, s_stg, e_stg, sem_n, sem_s, sem_e, sem_o):
        wid = lax.axis_index("s") * NC + lax.axis_index("c")
        ebase = wid * bpw

        pltpu.sync_copy(uidx.at[pl.ds(ebase, bpw)], idx_v.at[0])
        cu = pltpu.async_copy(utab_h.at[idx_v.at[0]], nu_v, sem_n)
        pltpu.sync_copy(iidx.at[pl.ds(ebase, bpw)], idx_v.at[1])
        ci = pltpu.async_copy(itab_h.at[idx_v.at[1]], ni_v, sem_n)
        cu.wait()
        ci.wait()

        for n_v, scr, emb, s_out, e_out in (
                (nu_v, uscr_h, uemb_h, su, eu), (ni_v, iscr_h, iemb_h, si, ei)):
            @pl.loop(0, ngroups)
            def _(g):
                p = lax.rem(g, 2)
                @pl.when(g >= 2)
                def _():
                    pltpu.make_async_copy(
                        s_stg.at[p], s_out.at[pl.ds(0, G * P)], sem_o).wait()
                    pltpu.make_async_copy(
                        e_stg.at[p], e_out.at[pl.ds(0, G * P)], sem_o).wait()
                for m in range(G):
                    e = g * G + m
                    pltpu.async_copy(
                        scr.at[n_v.at[e]], s_stg.at[p, pl.ds(m * P, P)], sem_s)
                    pltpu.async_copy(
                        emb.at[n_v.at[e]], e_stg.at[p, pl.ds(m * P, P)], sem_e)
                for m in range(G):
                    pltpu.make_async_copy(
                        scr.at[pl.ds(0, P)], s_stg.at[p, pl.ds(m * P, P)],
                        sem_s).wait()
                    pltpu.make_async_copy(
                        emb.at[pl.ds(0, P)], e_stg.at[p, pl.ds(m * P, P)],
                        sem_e).wait()
                rbase = (ebase + g * G) * P
                pltpu.async_copy(s_stg.at[p], s_out.at[pl.ds(rbase, G * P)],
                                 sem_o)
                pltpu.async_copy(e_stg.at[p], e_out.at[pl.ds(rbase, G * P)],
                                 sem_o)
            for _p in range(2):
                pltpu.make_async_copy(
                    s_stg.at[_p], s_out.at[pl.ds(0, G * P)], sem_o).wait()
                pltpu.make_async_copy(
                    e_stg.at[_p], e_out.at[pl.ds(0, G * P)], sem_o).wait()

    R = B * P
    call = pl.kernel(
        body,
        out_type=(
            jax.ShapeDtypeStruct((R, W), jnp.float32),
            jax.ShapeDtypeStruct((R, EMB), jnp.float32),
            jax.ShapeDtypeStruct((R, W), jnp.float32),
            jax.ShapeDtypeStruct((R, EMB), jnp.float32),
        ),
        mesh=_mesh(),
        scratch_types=[
            pltpu.VMEM((2, bpw), jnp.int32),
            pltpu.VMEM((bpw, P), jnp.int32),
            pltpu.VMEM((bpw, P), jnp.int32),
            pltpu.VMEM((2, G * P, W), jnp.float32),
            pltpu.VMEM((2, G * P, EMB), jnp.float32),
            pltpu.SemaphoreType.DMA,
            pltpu.SemaphoreType.DMA,
            pltpu.SemaphoreType.DMA,
            pltpu.SemaphoreType.DMA,
        ],
        compiler_params=pltpu.CompilerParams(use_tc_tiling_on_sc=False),
    )
    return call(user_idxs, item_idxs, utab, itab, uscr, iscr, uemb, iemb)


def _mlp_body(su_ref, eu_ref, si_ref, ei_ref, w1_ref, b1_ref, w2_ref, b2_ref,
              w3_ref, b3_ref, out_ref):
    f32 = jnp.float32
    w1 = w1_ref[...]
    gu = jnp.dot(eu_ref[...], w1[0:EMB, :], preferred_element_type=f32)
    gi = jnp.dot(ei_ref[...], w1[EMB:2 * EMB, :], preferred_element_type=f32)
    su = su_ref[...]
    si = si_ref[...]
    z8 = jnp.zeros((W - P, EMB), f32)
    evens, odds = [], []
    for e in range(BLK):
        fsu = su[e * HR:(e + 1) * HR, :]
        fsi = si[e * HR:(e + 1) * HR, :]
        gue = jnp.concatenate([gu[e * P:(e + 1) * P, :], z8], axis=0)
        gie = jnp.concatenate([gi[e * P:(e + 1) * P, :], z8], axis=0)
        evens.append(jnp.dot(fsu[:, 0:W], gue, preferred_element_type=f32)
                     + jnp.dot(fsi[:, 0:W], gie, preferred_element_type=f32))
        odds.append(jnp.dot(fsu[:, W:128], gue, preferred_element_type=f32)
                    + jnp.dot(fsi[:, W:128], gie, preferred_element_type=f32))
    pre1 = jnp.concatenate(evens + odds, axis=0)
    h1 = jnp.maximum(pre1 + b1_ref[...], 0.0)
    h2 = jnp.maximum(
        jnp.dot(h1, w2_ref[...], preferred_element_type=f32) + b2_ref[...], 0.0)
    z = jnp.dot(h2, w3_ref[...], preferred_element_type=f32) + b3_ref[...]
    o = jax.nn.sigmoid(z)
    half = BLK * HR
    t = lax.broadcasted_iota(jnp.int32, (2 * half, BLK), 0)
    c = lax.broadcasted_iota(jnp.int32, (2 * half, BLK), 1)
    th = t % half
    msk = ((th // HR == c) & (th % HR < (NEIGH + 1) // 2)).astype(f32)
    out_ref[0, 0, :] = jnp.sum(o * msk, axis=0) * (1.0 / NEIGH)


def _mlp_call(su, eu, si, ei, W1, b1, W2, b2, W3, b3):
    B = eu.shape[0] // P
    srows = BLK * HR
    erows = BLK * P
    out = pl.pallas_call(
        _mlp_body,
        grid=(B // BLK,),
        in_specs=[
            pl.BlockSpec((srows, 128), lambda i: (i, 0)),
            pl.BlockSpec((erows, EMB), lambda i: (i, 0)),
            pl.BlockSpec((srows, 128), lambda i: (i, 0)),
            pl.BlockSpec((erows, EMB), lambda i: (i, 0)),
            pl.BlockSpec((2 * EMB, EMB), lambda i: (0, 0)),
            pl.BlockSpec((1, EMB), lambda i: (0, 0)),
            pl.BlockSpec((EMB, 8), lambda i: (0, 0)),
            pl.BlockSpec((1, 8), lambda i: (0, 0)),
            pl.BlockSpec((8, 1), lambda i: (0, 0)),
            pl.BlockSpec((1, 1), lambda i: (0, 0)),
        ],
        out_specs=pl.BlockSpec((1, 1, BLK), lambda i: (i, 0, 0)),
        out_shape=jax.ShapeDtypeStruct((B // BLK, 1, BLK), jnp.float32),
    )(su, eu, si, ei, W1, b1.reshape(1, EMB), W2, b2.reshape(1, 8),
      W3, b3.reshape(1, 1))
    return out.reshape(B)


def kernel(user_idxs, item_idxs, user_idx_tensor, item_idx_tensor,
           user_scr_tensor, item_scr_tensor, user_emb, item_emb,
           W1, b1, W2, b2, W3, b3):
    utab = jnp.concatenate(
        [user_idx_tensor, user_idx_tensor[:, :P - NEIGH]], axis=1)
    itab = jnp.concatenate(
        [item_idx_tensor, item_idx_tensor[:, :P - NEIGH]], axis=1)
    padw = ((0, 0), (0, W - NEIGH))
    uscr = jnp.pad(user_scr_tensor, padw)
    iscr = jnp.pad(item_scr_tensor, padw)
    su, eu, si, ei = _gather_all(user_idxs, item_idxs, utab, itab,
                                 uscr, iscr, user_emb, item_emb)
    su = su.reshape(-1, 128)
    si = si.reshape(-1, 128)
    return _mlp_call(su, eu, si, ei, W1, b1, W2, b2, W3, b3)


# speedup vs baseline: 3.8352x; 1.0004x over previous
import jax
import jax.numpy as jnp
from jax import lax
from jax.experimental import pallas as pl
from jax.experimental.pallas import tpu as pltpu
from jax.experimental.pallas import tpu_sc as plsc

NC, NS = 2, 16
NW = NC * NS
NEIGH = 50
P = 56
W = 64
HR = P * W // 128
EMB = 16
G = 8
BLK = 64


def _mesh():
    return plsc.VectorSubcoreMesh(
        core_axis_name="c", subcore_axis_name="s", num_cores=NC, num_subcores=NS)


def _gather_all(user_idxs, item_idxs, utab, itab, uscr, iscr, uemb, iemb):
    B = user_idxs.shape[0]
    bpw = B // NW
    ngroups = bpw // G

    def body(uidx, iidx, utab_h, itab_h, uscr_h, iscr_h, uemb_h, iemb_h,
             su, eu, si, ei,
             idx_v, nu_v, ni_v, s_stg, e_stg, sem_n, sem_s, sem_e, sem_o):
        wid = lax.axis_index("s") * NC + lax.axis_index("c")
        ebase = wid * bpw

        pltpu.sync_copy(uidx.at[pl.ds(ebase, bpw)], idx_v.at[0])
        cu = pltpu.async_copy(utab_h.at[idx_v.at[0]], nu_v, sem_n)
        pltpu.sync_copy(iidx.at[pl.ds(ebase, bpw)], idx_v.at[1])
        ci = pltpu.async_copy(itab_h.at[idx_v.at[1]], ni_v, sem_n)
        cu.wait()
        ci.wait()

        for n_v, scr, emb, s_out, e_out in (
                (nu_v, uscr_h, uemb_h, su, eu), (ni_v, iscr_h, iemb_h, si, ei)):
            @pl.loop(0, ngroups)
            def _(g):
                p = lax.rem(g, 2)
                @pl.when(g >= 2)
                def _():
                    pltpu.make_async_copy(
                        s_stg.at[p], s_out.at[pl.ds(0, G * P)], sem_o).wait()
                    pltpu.make_async_copy(
                        e_stg.at[p], e_out.at[pl.ds(0, G * P)], sem_o).wait()
                for m in range(G):
                    e = g * G + m
                    pltpu.async_copy(
                        scr.at[n_v.at[e]], s_stg.at[p, pl.ds(m * P, P)], sem_s)
                    pltpu.async_copy(
                        emb.at[n_v.at[e]], e_stg.at[p, pl.ds(m * P, P)], sem_e)
                for m in range(G):
                    pltpu.make_async_copy(
                        scr.at[pl.ds(0, P)], s_stg.at[p, pl.ds(m * P, P)],
                        sem_s).wait()
                    pltpu.make_async_copy(
                        emb.at[pl.ds(0, P)], e_stg.at[p, pl.ds(m * P, P)],
                        sem_e).wait()
                rbase = (ebase + g * G) * P
                pltpu.async_copy(s_stg.at[p], s_out.at[pl.ds(rbase, G * P)],
                                 sem_o)
                pltpu.async_copy(e_stg.at[p], e_out.at[pl.ds(rbase, G * P)],
                                 sem_o)
            for _p in range(2):
                pltpu.make_async_copy(
                    s_stg.at[_p], s_out.at[pl.ds(0, G * P)], sem_o).wait()
                pltpu.make_async_copy(
                    e_stg.at[_p], e_out.at[pl.ds(0, G * P)], sem_o).wait()

    R = B * P
    call = pl.kernel(
        body,
        out_type=(
            jax.ShapeDtypeStruct((R, W), jnp.float32),
            jax.ShapeDtypeStruct((R, EMB), jnp.float32),
            jax.ShapeDtypeStruct((R, W), jnp.float32),
            jax.ShapeDtypeStruct((R, EMB), jnp.float32),
        ),
        mesh=_mesh(),
        scratch_types=[
            pltpu.VMEM((2, bpw), jnp.int32),
            pltpu.VMEM((bpw, P), jnp.int32),
            pltpu.VMEM((bpw, P), jnp.int32),
            pltpu.VMEM((2, G * P, W), jnp.float32),
            pltpu.VMEM((2, G * P, EMB), jnp.float32),
            pltpu.SemaphoreType.DMA,
            pltpu.SemaphoreType.DMA,
            pltpu.SemaphoreType.DMA,
            pltpu.SemaphoreType.DMA,
        ],
        compiler_params=pltpu.CompilerParams(use_tc_tiling_on_sc=False),
    )
    return call(user_idxs, item_idxs, utab, itab, uscr, iscr, uemb, iemb)


def _mlp_body(su_ref, eu_ref, si_ref, ei_ref, w1_ref, b1_ref, w2_ref, b2_ref,
              w3_ref, b3_ref, out_ref):
    f32 = jnp.float32
    w1 = w1_ref[...]
    gu = jnp.dot(eu_ref[...], w1[0:EMB, :], preferred_element_type=f32)
    gi = jnp.dot(ei_ref[...], w1[EMB:2 * EMB, :], preferred_element_type=f32)
    su = su_ref[...]
    si = si_ref[...]
    evens, odds = [], []
    for e in range(BLK):
        fsu = su[e * HR:(e + 1) * HR, :]
        fsi = si[e * HR:(e + 1) * HR, :]
        gue = gu[e * P:(e + 1) * P, :]
        gie = gi[e * P:(e + 1) * P, :]
        evens.append(jnp.dot(fsu[:, 0:P], gue, preferred_element_type=f32)
                     + jnp.dot(fsi[:, 0:P], gie, preferred_element_type=f32))
        odds.append(jnp.dot(fsu[:, W:W + P], gue, preferred_element_type=f32)
                    + jnp.dot(fsi[:, W:W + P], gie, preferred_element_type=f32))
    pre1 = jnp.concatenate(evens + odds, axis=0)
    h1 = jnp.maximum(pre1 + b1_ref[...], 0.0)
    h2 = jnp.maximum(
        jnp.dot(h1, w2_ref[...], preferred_element_type=f32) + b2_ref[...], 0.0)
    z = jnp.dot(h2, w3_ref[...], preferred_element_type=f32) + b3_ref[...]
    o = jax.nn.sigmoid(z)
    half = BLK * HR
    t = lax.broadcasted_iota(jnp.int32, (2 * half, BLK), 0)
    c = lax.broadcasted_iota(jnp.int32, (2 * half, BLK), 1)
    th = t % half
    msk = ((th // HR == c) & (th % HR < (NEIGH + 1) // 2)).astype(f32)
    out_ref[0, 0, :] = jnp.sum(o * msk, axis=0) * (1.0 / NEIGH)


def _mlp_call(su, eu, si, ei, W1, b1, W2, b2, W3, b3):
    B = eu.shape[0] // P
    srows = BLK * HR
    erows = BLK * P
    out = pl.pallas_call(
        _mlp_body,
        grid=(B // BLK,),
        in_specs=[
            pl.BlockSpec((srows, 128), lambda i: (i, 0)),
            pl.BlockSpec((erows, EMB), lambda i: (i, 0)),
            pl.BlockSpec((srows, 128), lambda i: (i, 0)),
            pl.BlockSpec((erows, EMB), lambda i: (i, 0)),
            pl.BlockSpec((2 * EMB, EMB), lambda i: (0, 0)),
            pl.BlockSpec((1, EMB), lambda i: (0, 0)),
            pl.BlockSpec((EMB, 8), lambda i: (0, 0)),
            pl.BlockSpec((1, 8), lambda i: (0, 0)),
            pl.BlockSpec((8, 1), lambda i: (0, 0)),
            pl.BlockSpec((1, 1), lambda i: (0, 0)),
        ],
        out_specs=pl.BlockSpec((1, 1, BLK), lambda i: (i, 0, 0)),
        out_shape=jax.ShapeDtypeStruct((B // BLK, 1, BLK), jnp.float32),
    )(su, eu, si, ei, W1, b1.reshape(1, EMB), W2, b2.reshape(1, 8),
      W3, b3.reshape(1, 1))
    return out.reshape(B)


def kernel(user_idxs, item_idxs, user_idx_tensor, item_idx_tensor,
           user_scr_tensor, item_scr_tensor, user_emb, item_emb,
           W1, b1, W2, b2, W3, b3):
    utab = jnp.concatenate(
        [user_idx_tensor, user_idx_tensor[:, :P - NEIGH]], axis=1)
    itab = jnp.concatenate(
        [item_idx_tensor, item_idx_tensor[:, :P - NEIGH]], axis=1)
    padw = ((0, 0), (0, W - NEIGH))
    uscr = jnp.pad(user_scr_tensor, padw)
    iscr = jnp.pad(item_scr_tensor, padw)
    su, eu, si, ei = _gather_all(user_idxs, item_idxs, utab, itab,
                                 uscr, iscr, user_emb, item_emb)
    su = su.reshape(-1, 128)
    si = si.reshape(-1, 128)
    return _mlp_call(su, eu, si, ei, W1, b1, W2, b2, W3, b3)
